# baseline (speedup 1.0000x reference)
import functools

import jax
import jax.numpy as jnp
from jax import lax
from jax.experimental import pallas as pl
from jax.experimental.pallas import tpu as pltpu
from jax.experimental.pallas import tpu_sc as plsc

EMBED = 768
SEQ = 2048
BATCH = 4
NTOK = BATCH * SEQ
NW = 32
POSW = SEQ // NW
WIN = 16
NWIN = POSW // WIN
LANES = 16
EMB_VECS = EMBED // LANES

_mesh = plsc.VectorSubcoreMesh(core_axis_name="c", subcore_axis_name="s")


@functools.partial(
    pl.kernel,
    out_type=jax.ShapeDtypeStruct((NTOK, EMBED), jnp.float32),
    mesh=_mesh,
    scratch_types=[
        pltpu.VMEM((BATCH, POSW), jnp.int32),
        pltpu.VMEM((2, BATCH, WIN, EMBED), jnp.float32),
        pltpu.VMEM((2, WIN, EMBED), jnp.float32),
        pltpu.SemaphoreType.DMA,
        (pltpu.SemaphoreType.DMA,) * 2,
        (pltpu.SemaphoreType.DMA,) * 2,
    ],
)
def _embed_add(ids_hbm, wte_hbm, wpe_hbm, out_hbm,
               idx_v, tok_v, pos_v, sem_idx, sem_in, sem_out):
    wid = lax.axis_index("s") * 2 + lax.axis_index("c")
    p0 = wid * POSW

    idx_copies = [
        pltpu.async_copy(ids_hbm.at[pl.ds(b * SEQ + p0, POSW)],
                         idx_v.at[b], sem_idx)
        for b in range(BATCH)
    ]
    for cp in idx_copies:
        cp.wait()

    def issue_in(w, slot):
        o = w * WIN
        copies = [
            pltpu.async_copy(wte_hbm.at[idx_v.at[b, pl.ds(o, WIN)]],
                             tok_v.at[slot, b], sem_in[slot])
            for b in range(BATCH)
        ]
        copies.append(
            pltpu.async_copy(wpe_hbm.at[pl.ds(p0 + o, WIN)],
                             pos_v.at[slot], sem_in[slot]))
        return copies

    def issue_out(w, slot):
        o = w * WIN
        return [
            pltpu.async_copy(tok_v.at[slot, b],
                             out_hbm.at[pl.ds(b * SEQ + p0 + o, WIN)],
                             sem_out[slot])
            for b in range(BATCH)
        ]

    out_copies = [None, None]
    in_flight = {0: issue_in(0, 0)}
    for w in range(NWIN):
        slot = w % 2
        for cp in in_flight.pop(w):
            cp.wait()

        def row_add(r):
            for k in range(EMB_VECS):
                sl = pl.ds(k * LANES, LANES)
                pv = pos_v[slot, r, sl]
                for b in range(BATCH):
                    plsc.addupdate(tok_v.at[slot, b, r, sl], pv)

        pl.loop(0, WIN, unroll=4)(row_add)
        out_copies[slot] = issue_out(w, slot)
        if w + 1 < NWIN:
            nslot = (w + 1) % 2
            if out_copies[nslot] is not None:
                for cp in out_copies[nslot]:
                    cp.wait()
            in_flight[w + 1] = issue_in(w + 1, nslot)
    for slot in range(2):
        if out_copies[slot] is not None:
            for cp in out_copies[slot]:
                cp.wait()


def kernel(input_ids, wte, wpe):
    b, s = input_ids.shape
    ids = input_ids.reshape(-1).astype(jnp.int32)
    out = _embed_add(ids, wte, wpe)
    return out.reshape(b, s, EMBED)


# speedup vs baseline: 1.1476x; 1.1476x over previous
import functools

import jax
import jax.numpy as jnp
from jax import lax
from jax.experimental import pallas as pl
from jax.experimental.pallas import tpu as pltpu
from jax.experimental.pallas import tpu_sc as plsc

EMBED = 768
SEQ = 2048
BATCH = 4
NTOK = BATCH * SEQ
NW = 32
POSW = SEQ // NW
WIN = 16
NWIN = POSW // WIN
LANES = 16
EMB_VECS = EMBED // LANES

_mesh = plsc.VectorSubcoreMesh(core_axis_name="c", subcore_axis_name="s")


@functools.partial(
    pl.kernel,
    out_type=jax.ShapeDtypeStruct((NTOK, EMBED), jnp.float32),
    mesh=_mesh,
    scratch_types=[
        pltpu.VMEM((BATCH, POSW), jnp.int32),
        pltpu.VMEM((2, BATCH, WIN, EMBED), jnp.float32),
        pltpu.VMEM((2, WIN, EMBED), jnp.float32),
        pltpu.SemaphoreType.DMA,
        (pltpu.SemaphoreType.DMA,) * 2,
        (pltpu.SemaphoreType.DMA,) * 2,
    ],
)
def _embed_add(ids_hbm, wte_hbm, wpe_hbm, out_hbm,
               idx_v, tok_v, pos_v, sem_idx, sem_in, sem_out):
    wid = lax.axis_index("s") * 2 + lax.axis_index("c")
    p0 = wid * POSW

    idx_copies = [
        pltpu.async_copy(ids_hbm.at[pl.ds(b * SEQ + p0, POSW)],
                         idx_v.at[b], sem_idx)
        for b in range(BATCH)
    ]
    for cp in idx_copies:
        cp.wait()

    def issue_in(w, slot):
        o = w * WIN
        copies = [
            pltpu.async_copy(wte_hbm.at[idx_v.at[b, pl.ds(o, WIN)]],
                             tok_v.at[slot, b], sem_in[slot])
            for b in range(BATCH)
        ]
        copies.append(
            pltpu.async_copy(wpe_hbm.at[pl.ds(p0 + o, WIN)],
                             pos_v.at[slot], sem_in[slot]))
        return copies

    def issue_out(w, slot):
        o = w * WIN
        return [
            pltpu.async_copy(tok_v.at[slot, b],
                             out_hbm.at[pl.ds(b * SEQ + p0 + o, WIN)],
                             sem_out[slot])
            for b in range(BATCH)
        ]

    out_copies = [None, None]
    in_flight = {0: issue_in(0, 0)}
    for w in range(NWIN):
        slot = w % 2
        for cp in in_flight.pop(w):
            cp.wait()

        def row_add(r):
            for k in range(EMB_VECS):
                sl = pl.ds(k * LANES, LANES)
                pv = pos_v[slot, r, sl]
                for b in range(BATCH):
                    plsc.addupdate(tok_v.at[slot, b, r, sl], pv)

        pl.loop(0, WIN, unroll=1)(row_add)
        out_copies[slot] = issue_out(w, slot)
        if w + 1 < NWIN:
            nslot = (w + 1) % 2
            if out_copies[nslot] is not None:
                for cp in out_copies[nslot]:
                    cp.wait()
            in_flight[w + 1] = issue_in(w + 1, nslot)
    for slot in range(2):
        if out_copies[slot] is not None:
            for cp in out_copies[slot]:
                cp.wait()


def kernel(input_ids, wte, wpe):
    b, s = input_ids.shape
    ids = input_ids.reshape(-1).astype(jnp.int32)
    out = _embed_add(ids, wte, wpe)
    return out.reshape(b, s, EMBED)


# speedup vs baseline: 1.1632x; 1.0136x over previous
import functools

import jax
import jax.numpy as jnp
from jax import lax
from jax.experimental import pallas as pl
from jax.experimental.pallas import tpu as pltpu
from jax.experimental.pallas import tpu_sc as plsc

EMBED = 768
SEQ = 2048
BATCH = 4
NW = 32
POSW = SEQ // NW
WIN = 16
NWIN = POSW // WIN
LANES = 16
EMB_VECS = EMBED // LANES

_mesh = plsc.VectorSubcoreMesh(core_axis_name="c", subcore_axis_name="s")


@functools.partial(
    pl.kernel,
    out_type=jax.ShapeDtypeStruct((BATCH, SEQ, EMBED), jnp.float32),
    mesh=_mesh,
    scratch_types=[
        pltpu.VMEM((NWIN, BATCH * WIN), jnp.int32),
        pltpu.VMEM((2, BATCH * WIN, EMBED), jnp.float32),
        pltpu.VMEM((2, WIN, EMBED), jnp.float32),
        pltpu.SemaphoreType.DMA,
        (pltpu.SemaphoreType.DMA,) * 2,
        (pltpu.SemaphoreType.DMA,) * 2,
    ],
)
def _embed_add(ids_hbm, wte_hbm, wpe_hbm, out_hbm,
               idx_v, tok_v, pos_v, sem_idx, sem_in, sem_out):
    wid = lax.axis_index("s") * 2 + lax.axis_index("c")
    p0 = wid * POSW

    idx_copies = [
        pltpu.async_copy(ids_hbm.at[b, pl.ds(p0 + w * WIN, WIN)],
                         idx_v.at[w, pl.ds(b * WIN, WIN)], sem_idx)
        for w in range(NWIN) for b in range(BATCH)
    ]
    for cp in idx_copies:
        cp.wait()

    def issue_in(w, slot):
        return [
            pltpu.async_copy(wte_hbm.at[idx_v.at[w]], tok_v.at[slot],
                             sem_in[slot]),
            pltpu.async_copy(wpe_hbm.at[pl.ds(p0 + w * WIN, WIN)],
                             pos_v.at[slot], sem_in[slot]),
        ]

    def issue_out(w, slot):
        return [
            pltpu.async_copy(tok_v.at[slot, pl.ds(b * WIN, WIN)],
                             out_hbm.at[b, pl.ds(p0 + w * WIN, WIN), :],
                             sem_out[slot])
            for b in range(BATCH)
        ]

    out_copies = [None, None]
    in_flight = {0: issue_in(0, 0)}
    for w in range(NWIN):
        slot = w % 2
        for cp in in_flight.pop(w):
            cp.wait()

        def row_add(r):
            for k in range(EMB_VECS):
                sl = pl.ds(k * LANES, LANES)
                pv = pos_v[slot, r, sl]
                for b in range(BATCH):
                    plsc.addupdate(tok_v.at[slot, b * WIN + r, sl], pv)

        pl.loop(0, WIN, unroll=1)(row_add)
        out_copies[slot] = issue_out(w, slot)
        if w + 1 < NWIN:
            nslot = (w + 1) % 2
            if out_copies[nslot] is not None:
                for cp in out_copies[nslot]:
                    cp.wait()
            in_flight[w + 1] = issue_in(w + 1, nslot)
    for slot in range(2):
        if out_copies[slot] is not None:
            for cp in out_copies[slot]:
                cp.wait()


def kernel(input_ids, wte, wpe):
    ids = input_ids.astype(jnp.int32)
    return _embed_add(ids, wte, wpe)


# speedup vs baseline: 1.4051x; 1.2080x over previous
import functools

import jax
import jax.numpy as jnp
from jax import lax
from jax.experimental import pallas as pl
from jax.experimental.pallas import tpu as pltpu
from jax.experimental.pallas import tpu_sc as plsc

EMBED = 768
SEQ = 2048
BATCH = 4
NW = 32
POSW = SEQ // NW
WIN = 16
NWIN = POSW // WIN
LANES = 16
EMB_VECS = EMBED // LANES

_mesh = plsc.VectorSubcoreMesh(core_axis_name="c", subcore_axis_name="s")


@functools.partial(
    pl.kernel,
    out_type=jax.ShapeDtypeStruct((BATCH, SEQ, EMBED), jnp.float32),
    mesh=_mesh,
    scratch_types=[
        pltpu.VMEM((NWIN, BATCH * WIN), jnp.int32),
        pltpu.VMEM((2, BATCH * WIN, EMBED), jnp.float32),
        pltpu.VMEM((2, WIN, EMBED), jnp.float32),
        pltpu.SemaphoreType.DMA,
        (pltpu.SemaphoreType.DMA,) * 2,
        (pltpu.SemaphoreType.DMA,) * 2,
    ],
)
def _embed_add(ids_hbm, wte_hbm, wpe_hbm, out_hbm,
               idx_v, tok_v, pos_v, sem_idx, sem_in, sem_out):
    wid = lax.axis_index("s") * 2 + lax.axis_index("c")
    p0 = wid * POSW

    idx_copies = [
        pltpu.async_copy(ids_hbm.at[b, pl.ds(p0 + w * WIN, WIN)],
                         idx_v.at[w, pl.ds(b * WIN, WIN)], sem_idx)
        for w in range(NWIN) for b in range(BATCH)
    ]
    for cp in idx_copies:
        cp.wait()

    def issue_in(w, slot):
        return [
            pltpu.async_copy(wte_hbm.at[idx_v.at[w]], tok_v.at[slot],
                             sem_in[slot]),
            pltpu.async_copy(wpe_hbm.at[pl.ds(p0 + w * WIN, WIN)],
                             pos_v.at[slot], sem_in[slot]),
        ]

    def issue_out(w, slot):
        return [
            pltpu.async_copy(tok_v.at[slot, pl.ds(b * WIN, WIN)],
                             out_hbm.at[b, pl.ds(p0 + w * WIN, WIN), :],
                             sem_out[slot])
            for b in range(BATCH)
        ]

    out_copies = [None, None]
    in_flight = {0: issue_in(0, 0)}
    for w in range(NWIN):
        slot = w % 2
        for cp in in_flight.pop(w):
            cp.wait()
        if w + 1 < NWIN:
            nslot = (w + 1) % 2
            if out_copies[nslot] is not None:
                for cp in out_copies[nslot]:
                    cp.wait()
            in_flight[w + 1] = issue_in(w + 1, nslot)

        def row_add(r):
            for k in range(EMB_VECS):
                sl = pl.ds(k * LANES, LANES)
                pv = pos_v[slot, r, sl]
                for b in range(BATCH):
                    plsc.addupdate(tok_v.at[slot, b * WIN + r, sl], pv)

        pl.loop(0, WIN, unroll=1)(row_add)
        out_copies[slot] = issue_out(w, slot)
    for slot in range(2):
        if out_copies[slot] is not None:
            for cp in out_copies[slot]:
                cp.wait()


def kernel(input_ids, wte, wpe):
    ids = input_ids.astype(jnp.int32)
    return _embed_add(ids, wte, wpe)
